# baseline (speedup 1.0000x reference)
import functools

import jax
import jax.numpy as jnp
from jax import lax
from jax.experimental import pallas as pl
from jax.experimental.pallas import tpu as pltpu
from jax.experimental.pallas import tpu_sc as plsc

B = 4096
F = 26
V = 100000
E = 16
L1 = 256
L2 = 64

NCORE = 2
NSUB = 16
FPC = F // NCORE
VRANGE = V // NSUB
VMAIN = 6400
VBLOCK = 6432
SCHUNK = 64
EPAD = 128
NOUT = B * F + 16
DUMP = B * F


def _floor128(x):
    return (x // 128) * 128


@functools.cache
def _make_sc_gather():
    mesh = plsc.VectorSubcoreMesh(core_axis_name="c", subcore_axis_name="s")
    return functools.partial(
        pl.kernel,
        out_type=jax.ShapeDtypeStruct((NOUT, EPAD), jnp.float32),
        mesh=mesh,
        scratch_types=[
            pltpu.VMEM((E, VBLOCK), jnp.float32),
            pltpu.VMEM((B,), jnp.int32),
            pltpu.VMEM((2, SCHUNK, EPAD), jnp.float32),
            pltpu.VMEM((2, SCHUNK), jnp.int32),
            pltpu.SemaphoreType.DMA,
            pltpu.SemaphoreType.DMA,
        ],
        compiler_params=pltpu.CompilerParams(needs_layout_passes=False),
    )(_sc_gather_body)


def _sc_gather_body(tab_hbm, idx_hbm, out_hbm, block_v, idx_v, stage_v,
                    didx_v, sem0, sem1):
    c = lax.axis_index("c")
    s = lax.axis_index("s")
    lane = lax.iota(jnp.int32, 16)

    lo = s * VRANGE
    load_lo = jnp.where(s == NSUB - 1, V - VBLOCK, _floor128(lo))
    load_lo = pl.multiple_of(load_lo, 128)

    def _field(k, carry):
        f = c * FPC + k

        pltpu.sync_copy(tab_hbm.at[f, :, pl.ds(load_lo, VMAIN)],
                        block_v.at[:, pl.ds(0, VMAIN)])

        @pl.when(s == NSUB - 1)
        def _():
            pltpu.sync_copy(tab_hbm.at[f, :, pl.ds(V - 32, 32)],
                            block_v.at[:, pl.ds(VMAIN, 32)])

        pltpu.sync_copy(idx_hbm.at[pl.ds(f * B, B)], idx_v)

        def _chunk(nc, cc):
            buf = nc % 2
            sem = [sem0, sem1]
            @pl.when(nc >= 2)
            def _():
                for b in range(2):
                    @pl.when(buf == b)
                    def _():
                        pltpu.make_async_copy(
                            stage_v.at[b], out_hbm.at[didx_v.at[b]],
                            sem[b]).wait()

            for gg in range(SCHUNK // 16):
                base_b = nc * SCHUNK + gg * 16
                v = idx_v[pl.ds(base_b, 16)]
                mask = (v >= lo) & (v < lo + VRANGE)
                cols = jnp.where(mask, v - load_lo, 0)
                rows = gg * 16 + lane
                for e in range(E):
                    erow = jnp.full((16,), e, jnp.int32)
                    got = plsc.load_gather(block_v, [erow, cols], mask=mask)
                    plsc.store_scatter(stage_v.at[buf], [rows, erow], got,
                                       mask=mask)
                dest = jnp.where(mask, (base_b + lane) * F + f, DUMP)
                didx_v[buf, pl.ds(gg * 16, 16)] = dest

            for b in range(2):
                @pl.when(buf == b)
                def _():
                    pltpu.async_copy(stage_v.at[b], out_hbm.at[didx_v.at[b]],
                                     sem[b])
            return cc

        lax.fori_loop(0, B // SCHUNK, _chunk, 0)
        for b in range(2):
            pltpu.make_async_copy(stage_v.at[b], out_hbm.at[didx_v.at[b]],
                                  [sem0, sem1][b]).wait()
        return carry

    lax.fori_loop(0, FPC, _field, 0)


def _dense_body(x_ref, w1_ref, b1_ref, w2_ref, b2_ref, p_ref, out_ref):
    x = x_ref[...]
    h1 = jnp.dot(x, w1_ref[...], preferred_element_type=jnp.float32)
    h1 = jnp.maximum(h1 + b1_ref[...], 0.0)
    ux = jnp.dot(h1, w2_ref[...], preferred_element_type=jnp.float32)
    ux = jnp.maximum(ux + b2_ref[...], 0.0)
    mx_ = jnp.dot(ux, p_ref[...], preferred_element_type=jnp.float32)
    m = jnp.max(mx_, axis=-1, keepdims=True)
    ex = jnp.exp(mx_ - m)
    mx = (jnp.float32(F) * ex) / jnp.sum(ex, axis=-1, keepdims=True)

    j_ids = lax.broadcasted_iota(jnp.int32, (F, F * E), 1)
    f_ids = lax.broadcasted_iota(jnp.int32, (F, F * E), 0)
    st = jnp.where(j_ids // E == f_ids, 1.0, 0.0).astype(jnp.float32)
    mxr = jnp.dot(mx, st, preferred_element_type=jnp.float32)

    a = mxr * x
    jj = lax.broadcasted_iota(jnp.int32, (F * E, E), 0)
    ee = lax.broadcasted_iota(jnp.int32, (F * E, E), 1)
    sm = jnp.where(jj % E == ee, 1.0, 0.0).astype(jnp.float32)
    sum_v = jnp.dot(a, sm, preferred_element_type=jnp.float32)
    sum_sq = jnp.dot(a * a, sm, preferred_element_type=jnp.float32)
    out_ref[...] = 0.5 * jnp.sum(sum_v * sum_v - sum_sq, axis=-1,
                                 keepdims=True)


_BB = 1024


def _dense(x, W1, b1, W2, b2, P):
    return pl.pallas_call(
        _dense_body,
        grid=(B // _BB,),
        in_specs=[
            pl.BlockSpec((_BB, F * E), lambda i: (i, 0)),
            pl.BlockSpec((F * E, L1), lambda i: (0, 0)),
            pl.BlockSpec((1, L1), lambda i: (0, 0)),
            pl.BlockSpec((L1, L2), lambda i: (0, 0)),
            pl.BlockSpec((1, L2), lambda i: (0, 0)),
            pl.BlockSpec((L2, F), lambda i: (0, 0)),
        ],
        out_specs=pl.BlockSpec((_BB, 1), lambda i: (i, 0)),
        out_shape=jax.ShapeDtypeStruct((B, 1), jnp.float32),
    )(x, W1, b1, W2, b2, P)


def kernel(sparse, dense, tables, W1, b1, W2, b2, P):
    tab = tables.transpose(0, 2, 1)
    idx = sparse.T.reshape(F * B)
    rows = _make_sc_gather()(tab, idx)
    x = rows[:B * F, :E].reshape(B, F * E)
    return _dense(x, W1, b1.reshape(1, L1), W2, b2.reshape(1, L2), P)


# speedup vs baseline: 131.7433x; 131.7433x over previous
import functools

import jax
import jax.numpy as jnp
from jax import lax
from jax.experimental import pallas as pl
from jax.experimental.pallas import tpu as pltpu
from jax.experimental.pallas import tpu_sc as plsc

B = 4096
F = 26
V = 100000
E = 16
L1 = 256
L2 = 64

NCORE = 2
NSUB = 16
FPC = F // NCORE
VRANGE = V // NSUB
VMAIN = 6400
VBLOCK = 6432
SCHUNK = 64
EPAD = 128
DUMP = B * F
NOUT = B * F + NCORE * NSUB * SCHUNK


def _floor128(x):
    return (x // 128) * 128


@functools.cache
def _make_sc_gather():
    mesh = plsc.VectorSubcoreMesh(core_axis_name="c", subcore_axis_name="s")
    return functools.partial(
        pl.kernel,
        out_type=jax.ShapeDtypeStruct((NOUT, EPAD), jnp.float32),
        mesh=mesh,
        scratch_types=[
            pltpu.VMEM((E, VBLOCK), jnp.float32),
            pltpu.VMEM((B,), jnp.int32),
            pltpu.VMEM((2, SCHUNK, EPAD), jnp.float32),
            pltpu.VMEM((2, SCHUNK), jnp.int32),
            pltpu.SemaphoreType.DMA,
        ],
        compiler_params=pltpu.CompilerParams(needs_layout_passes=False),
    )(_sc_gather_body)


def _sc_gather_body(tab_hbm, idx_hbm, out_hbm, block_v, idx_v, stage_v,
                    didx_v, sem0):
    c = lax.axis_index("c")
    s = lax.axis_index("s")
    lane = lax.iota(jnp.int32, 16)

    lo = s * VRANGE
    load_lo = jnp.where(s == NSUB - 1, V - VBLOCK, _floor128(lo))
    load_lo = pl.multiple_of(load_lo, 128)

    def _field(k, carry):
        f = c * FPC + k

        pltpu.sync_copy(tab_hbm.at[f, :, pl.ds(load_lo, VMAIN)],
                        block_v.at[:, pl.ds(0, VMAIN)])

        @pl.when(s == NSUB - 1)
        def _():
            pltpu.sync_copy(tab_hbm.at[f, :, pl.ds(V - 32, 32)],
                            block_v.at[:, pl.ds(VMAIN, 32)])

        pltpu.sync_copy(idx_hbm.at[pl.ds(f * B, B)], idx_v)

        wid = c * NSUB + s
        dump_base = DUMP + wid * SCHUNK

        def _group(g, count):
            base_b = g * 16
            v = idx_v[pl.ds(base_b, 16)]
            mask = (v >= lo) & (v < lo + VRANGE)
            cols = jnp.where(mask, v - load_lo, 0)
            inc = plsc.all_reduce_population_count(mask)[0]
            pos = count + plsc.cumsum(mask.astype(jnp.int32)) - 1
            bufv = (pos // SCHUNK) % 2
            rowv = pos % SCHUNK
            for e in range(E):
                erow = jnp.full((16,), e, jnp.int32)
                got = plsc.load_gather(block_v, [erow, cols], mask=mask)
                plsc.store_scatter(stage_v, [bufv, rowv, erow], got,
                                   mask=mask)
            dest = (base_b + lane) * F + f
            plsc.store_scatter(didx_v, [bufv, rowv], dest, mask=mask)
            count_new = count + inc

            @pl.when(count_new // SCHUNK > count // SCHUNK)
            def _():
                cbm = (count // SCHUNK) % 2
                pltpu.async_copy(stage_v.at[cbm], out_hbm.at[didx_v.at[cbm]],
                                 sem0).wait()
            return count_new

        count = lax.fori_loop(0, B // 16, _group, 0)

        rem = count % SCHUNK

        @pl.when(rem > 0)
        def _():
            pbm = (count // SCHUNK) % 2
            pbv = jnp.zeros((16,), jnp.int32) + pbm
            for j in range(SCHUNK // 16):
                sl = j * 16 + lane
                plsc.store_scatter(didx_v, [pbv, sl], dump_base + sl,
                                   mask=sl >= rem)
            pltpu.async_copy(stage_v.at[pbm], out_hbm.at[didx_v.at[pbm]],
                             sem0).wait()
        return carry

    lax.fori_loop(0, FPC, _field, 0)


def _dense_body(x_ref, w1_ref, b1_ref, w2_ref, b2_ref, p_ref, out_ref):
    x = x_ref[...]
    h1 = jnp.dot(x, w1_ref[...], preferred_element_type=jnp.float32)
    h1 = jnp.maximum(h1 + b1_ref[...], 0.0)
    ux = jnp.dot(h1, w2_ref[...], preferred_element_type=jnp.float32)
    ux = jnp.maximum(ux + b2_ref[...], 0.0)
    mx_ = jnp.dot(ux, p_ref[...], preferred_element_type=jnp.float32)
    m = jnp.max(mx_, axis=-1, keepdims=True)
    ex = jnp.exp(mx_ - m)
    mx = (jnp.float32(F) * ex) / jnp.sum(ex, axis=-1, keepdims=True)

    j_ids = lax.broadcasted_iota(jnp.int32, (F, F * E), 1)
    f_ids = lax.broadcasted_iota(jnp.int32, (F, F * E), 0)
    st = jnp.where(j_ids // E == f_ids, 1.0, 0.0).astype(jnp.float32)
    mxr = jnp.dot(mx, st, preferred_element_type=jnp.float32)

    a = mxr * x
    jj = lax.broadcasted_iota(jnp.int32, (F * E, E), 0)
    ee = lax.broadcasted_iota(jnp.int32, (F * E, E), 1)
    sm = jnp.where(jj % E == ee, 1.0, 0.0).astype(jnp.float32)
    sum_v = jnp.dot(a, sm, preferred_element_type=jnp.float32)
    sum_sq = jnp.dot(a * a, sm, preferred_element_type=jnp.float32)
    out_ref[...] = 0.5 * jnp.sum(sum_v * sum_v - sum_sq, axis=-1,
                                 keepdims=True)


_BB = 1024


def _dense(x, W1, b1, W2, b2, P):
    return pl.pallas_call(
        _dense_body,
        grid=(B // _BB,),
        in_specs=[
            pl.BlockSpec((_BB, F * E), lambda i: (i, 0)),
            pl.BlockSpec((F * E, L1), lambda i: (0, 0)),
            pl.BlockSpec((1, L1), lambda i: (0, 0)),
            pl.BlockSpec((L1, L2), lambda i: (0, 0)),
            pl.BlockSpec((1, L2), lambda i: (0, 0)),
            pl.BlockSpec((L2, F), lambda i: (0, 0)),
        ],
        out_specs=pl.BlockSpec((_BB, 1), lambda i: (i, 0)),
        out_shape=jax.ShapeDtypeStruct((B, 1), jnp.float32),
    )(x, W1, b1, W2, b2, P)


def kernel(sparse, dense, tables, W1, b1, W2, b2, P):
    tab = tables.transpose(0, 2, 1)
    idx = sparse.T.reshape(F * B)
    rows = _make_sc_gather()(tab, idx)
    x = rows[:B * F, :E].reshape(B, F * E)
    return _dense(x, W1, b1.reshape(1, L1), W2, b2.reshape(1, L2), P)


# speedup vs baseline: 234.8666x; 1.7828x over previous
import functools

import jax
import jax.numpy as jnp
from jax import lax
from jax.experimental import pallas as pl
from jax.experimental.pallas import tpu as pltpu
from jax.experimental.pallas import tpu_sc as plsc

B = 4096
F = 26
V = 100000
E = 16
L1 = 256
L2 = 64

NCORE = 2
NSUB = 16
FPC = F // NCORE
VRANGE = V // NSUB
VMAIN = 6400
VBLOCK = 6432
SCHUNK = 64
EPAD = 128
DUMP = B * F
NOUT = B * F + NCORE * NSUB * SCHUNK


def _floor128(x):
    return (x // 128) * 128


@functools.cache
def _make_sc_gather():
    mesh = plsc.VectorSubcoreMesh(core_axis_name="c", subcore_axis_name="s")
    return functools.partial(
        pl.kernel,
        out_type=jax.ShapeDtypeStruct((NOUT, EPAD), jnp.float32),
        mesh=mesh,
        scratch_types=[
            pltpu.VMEM((E, VBLOCK), jnp.float32),
            pltpu.VMEM((B,), jnp.int32),
            pltpu.VMEM((B + 32,), jnp.int32),
            pltpu.VMEM((2, SCHUNK, EPAD), jnp.float32),
            pltpu.VMEM((2, SCHUNK), jnp.int32),
            pltpu.SemaphoreType.DMA,
        ],
        compiler_params=pltpu.CompilerParams(needs_layout_passes=False),
    )(_sc_gather_body)


def _sc_gather_body(tab_hbm, idx_hbm, out_hbm, block_v, idx_v, wl_v, stage_v,
                    didx_v, sem0):
    c = lax.axis_index("c")
    s = lax.axis_index("s")
    lane = lax.iota(jnp.int32, 16)

    lo = s * VRANGE
    load_lo = jnp.where(s == NSUB - 1, V - VBLOCK, _floor128(lo))
    load_lo = pl.multiple_of(load_lo, 128)

    def _field(k, carry):
        f = c * FPC + k

        pltpu.sync_copy(tab_hbm.at[f, :, pl.ds(load_lo, VMAIN)],
                        block_v.at[:, pl.ds(0, VMAIN)])

        @pl.when(s == NSUB - 1)
        def _():
            pltpu.sync_copy(tab_hbm.at[f, :, pl.ds(V - 32, 32)],
                            block_v.at[:, pl.ds(VMAIN, 32)])

        pltpu.sync_copy(idx_hbm.at[pl.ds(f * B, B)], idx_v)

        wid = c * NSUB + s
        dump_base = DUMP + wid * SCHUNK

        def _scan(g, count):
            base_b = g * 16
            v = idx_v[pl.ds(base_b, 16)]
            mask = (v >= lo) & (v < lo + VRANGE)
            inc = plsc.all_reduce_population_count(mask)[0]
            plsc.store_compressed(wl_v.at[pl.ds(count, 16)], base_b + lane,
                                  mask=mask)
            return count + inc

        count = lax.fori_loop(0, B // 16, _scan, 0)

        nb = (count + SCHUNK - 1) // SCHUNK

        def _blk(blk, cc):
            buf = blk % 2
            @pl.when(blk >= 2)
            def _():
                pltpu.make_async_copy(stage_v.at[buf],
                                      out_hbm.at[didx_v.at[buf]],
                                      sem0).wait()
            for gg in range(SCHUNK // 16):
                slot = blk * SCHUNK + gg * 16 + lane
                sm = slot < count
                b = wl_v[pl.ds(blk * SCHUNK + gg * 16, 16)]
                b = jnp.where(sm, b, 0)
                vv = plsc.load_gather(idx_v, [b])
                cols = jnp.clip(vv - load_lo, 0, VBLOCK - 1)
                rows = gg * 16 + lane
                for e in range(E):
                    erow = jnp.full((16,), e, jnp.int32)
                    got = plsc.load_gather(block_v, [erow, cols])
                    plsc.store_scatter(stage_v.at[buf], [rows, erow], got)
                dest = jnp.where(sm, b * F + f, dump_base + gg * 16 + lane)
                didx_v[buf, pl.ds(gg * 16, 16)] = dest
            pltpu.async_copy(stage_v.at[buf], out_hbm.at[didx_v.at[buf]],
                             sem0)
            return cc

        lax.fori_loop(0, nb, _blk, 0)

        @pl.when(nb >= 2)
        def _():
            bb = (nb - 2) % 2
            pltpu.make_async_copy(stage_v.at[bb], out_hbm.at[didx_v.at[bb]],
                                  sem0).wait()

        @pl.when(nb >= 1)
        def _():
            bb = (nb - 1) % 2
            pltpu.make_async_copy(stage_v.at[bb], out_hbm.at[didx_v.at[bb]],
                                  sem0).wait()
        return carry

    lax.fori_loop(0, FPC, _field, 0)


def _dense_body(x_ref, w1_ref, b1_ref, w2_ref, b2_ref, p_ref, out_ref):
    x = x_ref[...]
    h1 = jnp.dot(x, w1_ref[...], preferred_element_type=jnp.float32)
    h1 = jnp.maximum(h1 + b1_ref[...], 0.0)
    ux = jnp.dot(h1, w2_ref[...], preferred_element_type=jnp.float32)
    ux = jnp.maximum(ux + b2_ref[...], 0.0)
    mx_ = jnp.dot(ux, p_ref[...], preferred_element_type=jnp.float32)
    m = jnp.max(mx_, axis=-1, keepdims=True)
    ex = jnp.exp(mx_ - m)
    mx = (jnp.float32(F) * ex) / jnp.sum(ex, axis=-1, keepdims=True)

    j_ids = lax.broadcasted_iota(jnp.int32, (F, F * E), 1)
    f_ids = lax.broadcasted_iota(jnp.int32, (F, F * E), 0)
    st = jnp.where(j_ids // E == f_ids, 1.0, 0.0).astype(jnp.float32)
    mxr = jnp.dot(mx, st, preferred_element_type=jnp.float32)

    a = mxr * x
    jj = lax.broadcasted_iota(jnp.int32, (F * E, E), 0)
    ee = lax.broadcasted_iota(jnp.int32, (F * E, E), 1)
    sm = jnp.where(jj % E == ee, 1.0, 0.0).astype(jnp.float32)
    sum_v = jnp.dot(a, sm, preferred_element_type=jnp.float32)
    sum_sq = jnp.dot(a * a, sm, preferred_element_type=jnp.float32)
    out_ref[...] = 0.5 * jnp.sum(sum_v * sum_v - sum_sq, axis=-1,
                                 keepdims=True)


_BB = 1024


def _dense(x, W1, b1, W2, b2, P):
    return pl.pallas_call(
        _dense_body,
        grid=(B // _BB,),
        in_specs=[
            pl.BlockSpec((_BB, F * E), lambda i: (i, 0)),
            pl.BlockSpec((F * E, L1), lambda i: (0, 0)),
            pl.BlockSpec((1, L1), lambda i: (0, 0)),
            pl.BlockSpec((L1, L2), lambda i: (0, 0)),
            pl.BlockSpec((1, L2), lambda i: (0, 0)),
            pl.BlockSpec((L2, F), lambda i: (0, 0)),
        ],
        out_specs=pl.BlockSpec((_BB, 1), lambda i: (i, 0)),
        out_shape=jax.ShapeDtypeStruct((B, 1), jnp.float32),
    )(x, W1, b1, W2, b2, P)


def kernel(sparse, dense, tables, W1, b1, W2, b2, P):
    tab = tables.transpose(0, 2, 1)
    idx = sparse.T.reshape(F * B)
    rows = _make_sc_gather()(tab, idx)
    x = rows[:B * F, :E].reshape(B, F * E)
    return _dense(x, W1, b1.reshape(1, L1), W2, b2.reshape(1, L2), P)


# speedup vs baseline: 274.9148x; 1.1705x over previous
import functools

import jax
import jax.numpy as jnp
from jax import lax
from jax.experimental import pallas as pl
from jax.experimental.pallas import tpu as pltpu
from jax.experimental.pallas import tpu_sc as plsc

B = 4096
F = 26
V = 100000
E = 16
L1 = 256
L2 = 64

NCORE = 2
NSUB = 16
FPC = F // NCORE
VRANGE = V // NSUB
VMAIN = 6400
VBLOCK = 6432
SCHUNK = 64
EPAD = 128
DUMP = B * F
NOUT = B * F + NCORE * NSUB * SCHUNK


def _floor128(x):
    return (x // 128) * 128


@functools.cache
def _make_sc_gather():
    mesh = plsc.VectorSubcoreMesh(core_axis_name="c", subcore_axis_name="s")
    return functools.partial(
        pl.kernel,
        out_type=jax.ShapeDtypeStruct((NOUT, EPAD), jnp.float32),
        mesh=mesh,
        scratch_types=[
            pltpu.VMEM((E, VBLOCK), jnp.float32),
            pltpu.VMEM((B,), jnp.int32),
            pltpu.VMEM((B + 32,), jnp.int32),
            pltpu.VMEM((2, SCHUNK, EPAD), jnp.float32),
            pltpu.VMEM((2, SCHUNK), jnp.int32),
            pltpu.SemaphoreType.DMA,
            pltpu.SemaphoreType.DMA,
        ],
        compiler_params=pltpu.CompilerParams(needs_layout_passes=False),
    )(_sc_gather_body)


def _sc_gather_body(tab_hbm, idx_hbm, out_hbm, block_v, idx_v, wl_v, stage_v,
                    didx_v, sem0, sem_slab):
    c = lax.axis_index("c")
    s = lax.axis_index("s")
    lane = lax.iota(jnp.int32, 16)

    lo = s * VRANGE
    load_lo = jnp.where(s == NSUB - 1, V - VBLOCK, _floor128(lo))
    load_lo = pl.multiple_of(load_lo, 128)

    def _field(k, carry):
        f = c * FPC + k

        slab_copy = pltpu.async_copy(tab_hbm.at[f, :, pl.ds(load_lo, VMAIN)],
                                     block_v.at[:, pl.ds(0, VMAIN)], sem_slab)
        pltpu.sync_copy(idx_hbm.at[pl.ds(f * B, B)], idx_v)

        wid = c * NSUB + s
        dump_base = DUMP + wid * SCHUNK

        def _scan(g4, count):
            for u in range(4):
                base_b = (g4 * 4 + u) * 16
                v = idx_v[pl.ds(base_b, 16)]
                mask = (v >= lo) & (v < lo + VRANGE)
                inc = plsc.all_reduce_population_count(mask)[0]
                plsc.store_compressed(wl_v.at[pl.ds(count, 16)],
                                      base_b + lane, mask=mask)
                count = count + inc
            return count

        count = lax.fori_loop(0, B // 64, _scan, 0)

        slab_copy.wait()

        @pl.when(s == NSUB - 1)
        def _():
            pltpu.sync_copy(tab_hbm.at[f, :, pl.ds(V - 32, 32)],
                            block_v.at[:, pl.ds(VMAIN, 32)])

        nb = (count + SCHUNK - 1) // SCHUNK

        def _blk(blk, cc):
            buf = blk % 2
            @pl.when(blk >= 2)
            def _():
                pltpu.make_async_copy(stage_v.at[buf],
                                      out_hbm.at[didx_v.at[buf]],
                                      sem0).wait()
            for gg in range(SCHUNK // 16):
                slot = blk * SCHUNK + gg * 16 + lane
                sm = slot < count
                b = wl_v[pl.ds(blk * SCHUNK + gg * 16, 16)]
                b = jnp.where(sm, b, 0)
                vv = plsc.load_gather(idx_v, [b])
                cols = jnp.clip(vv - load_lo, 0, VBLOCK - 1)
                rows = gg * 16 + lane
                for e in range(E):
                    erow = jnp.full((16,), e, jnp.int32)
                    got = plsc.load_gather(block_v, [erow, cols])
                    plsc.store_scatter(stage_v.at[buf], [rows, erow], got)
                dest = jnp.where(sm, b * F + f, dump_base + gg * 16 + lane)
                didx_v[buf, pl.ds(gg * 16, 16)] = dest
            pltpu.async_copy(stage_v.at[buf], out_hbm.at[didx_v.at[buf]],
                             sem0)
            return cc

        lax.fori_loop(0, nb, _blk, 0)

        @pl.when(nb >= 2)
        def _():
            bb = (nb - 2) % 2
            pltpu.make_async_copy(stage_v.at[bb], out_hbm.at[didx_v.at[bb]],
                                  sem0).wait()

        @pl.when(nb >= 1)
        def _():
            bb = (nb - 1) % 2
            pltpu.make_async_copy(stage_v.at[bb], out_hbm.at[didx_v.at[bb]],
                                  sem0).wait()
        return carry

    lax.fori_loop(0, FPC, _field, 0)


def _dense_body(x_ref, w1_ref, b1_ref, w2_ref, b2_ref, p_ref, out_ref):
    x = x_ref[...]
    h1 = jnp.dot(x, w1_ref[...], preferred_element_type=jnp.float32)
    h1 = jnp.maximum(h1 + b1_ref[...], 0.0)
    ux = jnp.dot(h1, w2_ref[...], preferred_element_type=jnp.float32)
    ux = jnp.maximum(ux + b2_ref[...], 0.0)
    mx_ = jnp.dot(ux, p_ref[...], preferred_element_type=jnp.float32)
    m = jnp.max(mx_, axis=-1, keepdims=True)
    ex = jnp.exp(mx_ - m)
    mx = (jnp.float32(F) * ex) / jnp.sum(ex, axis=-1, keepdims=True)

    j_ids = lax.broadcasted_iota(jnp.int32, (F, F * E), 1)
    f_ids = lax.broadcasted_iota(jnp.int32, (F, F * E), 0)
    st = jnp.where(j_ids // E == f_ids, 1.0, 0.0).astype(jnp.float32)
    mxr = jnp.dot(mx, st, preferred_element_type=jnp.float32)

    a = mxr * x
    jj = lax.broadcasted_iota(jnp.int32, (F * E, E), 0)
    ee = lax.broadcasted_iota(jnp.int32, (F * E, E), 1)
    sm = jnp.where(jj % E == ee, 1.0, 0.0).astype(jnp.float32)
    sum_v = jnp.dot(a, sm, preferred_element_type=jnp.float32)
    sum_sq = jnp.dot(a * a, sm, preferred_element_type=jnp.float32)
    out_ref[...] = 0.5 * jnp.sum(sum_v * sum_v - sum_sq, axis=-1,
                                 keepdims=True)


_BB = 1024


def _dense(x, W1, b1, W2, b2, P):
    return pl.pallas_call(
        _dense_body,
        grid=(B // _BB,),
        in_specs=[
            pl.BlockSpec((_BB, F * E), lambda i: (i, 0)),
            pl.BlockSpec((F * E, L1), lambda i: (0, 0)),
            pl.BlockSpec((1, L1), lambda i: (0, 0)),
            pl.BlockSpec((L1, L2), lambda i: (0, 0)),
            pl.BlockSpec((1, L2), lambda i: (0, 0)),
            pl.BlockSpec((L2, F), lambda i: (0, 0)),
        ],
        out_specs=pl.BlockSpec((_BB, 1), lambda i: (i, 0)),
        out_shape=jax.ShapeDtypeStruct((B, 1), jnp.float32),
    )(x, W1, b1, W2, b2, P)


def kernel(sparse, dense, tables, W1, b1, W2, b2, P):
    tab = tables.transpose(0, 2, 1)
    idx = sparse.T.reshape(F * B)
    rows = _make_sc_gather()(tab, idx)
    x = rows[:B * F, :E].reshape(B, F * E)
    return _dense(x, W1, b1.reshape(1, L1), W2, b2.reshape(1, L2), P)


# speedup vs baseline: 300.0625x; 1.0915x over previous
import functools

import jax
import jax.numpy as jnp
from jax import lax
from jax.experimental import pallas as pl
from jax.experimental.pallas import tpu as pltpu
from jax.experimental.pallas import tpu_sc as plsc

B = 4096
F = 26
V = 100000
E = 16
L1 = 256
L2 = 64

NCORE = 2
NSUB = 16
FPC = F // NCORE
VRANGE = V // NSUB
VMAIN = 6400
VBLOCK = 6432
SCHUNK = 64
EPAD = 128
NOUT = B * F


def _floor128(x):
    return (x // 128) * 128


@functools.cache
def _make_sc_gather():
    mesh = plsc.VectorSubcoreMesh(core_axis_name="c", subcore_axis_name="s")
    return functools.partial(
        pl.kernel,
        out_type=jax.ShapeDtypeStruct((NOUT, EPAD), jnp.float32),
        mesh=mesh,
        scratch_types=[
            pltpu.VMEM((E, VBLOCK), jnp.float32),
            pltpu.VMEM((B,), jnp.int32),
            pltpu.VMEM((B + 32,), jnp.int32),
            pltpu.VMEM((2, SCHUNK, EPAD), jnp.float32),
            pltpu.VMEM((2, SCHUNK), jnp.int32),
            pltpu.SemaphoreType.DMA,
            pltpu.SemaphoreType.DMA,
        ],
        compiler_params=pltpu.CompilerParams(needs_layout_passes=False),
    )(_sc_gather_body)


def _sc_gather_body(tab_hbm, idx_hbm, out_hbm, block_v, idx_v, wl_v, stage_v,
                    didx_v, sem0, sem_slab):
    c = lax.axis_index("c")
    s = lax.axis_index("s")
    lane = lax.iota(jnp.int32, 16)

    lo = s * VRANGE
    load_lo = jnp.where(s == NSUB - 1, V - VBLOCK, _floor128(lo))
    load_lo = pl.multiple_of(load_lo, 128)

    def _field(k, carry):
        f = c * FPC + k

        slab_copy = pltpu.async_copy(tab_hbm.at[f, :, pl.ds(load_lo, VMAIN)],
                                     block_v.at[:, pl.ds(0, VMAIN)], sem_slab)
        pltpu.sync_copy(idx_hbm.at[pl.ds(f * B, B)], idx_v)

        def _scan(g4, count):
            for u in range(4):
                base_b = (g4 * 4 + u) * 16
                v = idx_v[pl.ds(base_b, 16)]
                mask = (v >= lo) & (v < lo + VRANGE)
                inc = plsc.all_reduce_population_count(mask)[0]
                plsc.store_compressed(wl_v.at[pl.ds(count, 16)],
                                      base_b + lane, mask=mask)
                count = count + inc
            return count

        count = lax.fori_loop(0, B // 64, _scan, 0)

        slab_copy.wait()

        @pl.when(s == NSUB - 1)
        def _():
            pltpu.sync_copy(tab_hbm.at[f, :, pl.ds(V - 32, 32)],
                            block_v.at[:, pl.ds(VMAIN, 32)])

        nb = (count + SCHUNK - 1) // SCHUNK

        def _blk(blk, cc):
            buf = blk % 2
            @pl.when(blk >= 2)
            def _():
                pltpu.make_async_copy(stage_v.at[buf],
                                      out_hbm.at[didx_v.at[buf]],
                                      sem0).wait()
            b0 = wl_v[pl.ds(blk * SCHUNK, 16)][0]
            for gg in range(SCHUNK // 16):
                slot = blk * SCHUNK + gg * 16 + lane
                sm = slot < count
                b = wl_v[pl.ds(blk * SCHUNK + gg * 16, 16)]
                b = jnp.where(sm, b, b0)
                vv = plsc.load_gather(idx_v, [b])
                cols = vv - load_lo
                rows = gg * 16 + lane
                for e in range(E):
                    erow = jnp.full((16,), e, jnp.int32)
                    got = plsc.load_gather(block_v, [erow, cols])
                    plsc.store_scatter(stage_v.at[buf], [rows, erow], got)
                didx_v[buf, pl.ds(gg * 16, 16)] = b * F + f
            pltpu.async_copy(stage_v.at[buf], out_hbm.at[didx_v.at[buf]],
                             sem0)
            return cc

        lax.fori_loop(0, nb, _blk, 0)

        @pl.when(nb >= 2)
        def _():
            bb = (nb - 2) % 2
            pltpu.make_async_copy(stage_v.at[bb], out_hbm.at[didx_v.at[bb]],
                                  sem0).wait()

        @pl.when(nb >= 1)
        def _():
            bb = (nb - 1) % 2
            pltpu.make_async_copy(stage_v.at[bb], out_hbm.at[didx_v.at[bb]],
                                  sem0).wait()
        return carry

    lax.fori_loop(0, FPC, _field, 0)


def _dense_body(x_ref, w1_ref, b1_ref, w2_ref, b2_ref, p_ref, out_ref):
    x = x_ref[...]
    h1 = jnp.dot(x, w1_ref[...], preferred_element_type=jnp.float32)
    h1 = jnp.maximum(h1 + b1_ref[...], 0.0)
    ux = jnp.dot(h1, w2_ref[...], preferred_element_type=jnp.float32)
    ux = jnp.maximum(ux + b2_ref[...], 0.0)
    mx_ = jnp.dot(ux, p_ref[...], preferred_element_type=jnp.float32)
    m = jnp.max(mx_, axis=-1, keepdims=True)
    ex = jnp.exp(mx_ - m)
    mx = (jnp.float32(F) * ex) / jnp.sum(ex, axis=-1, keepdims=True)

    j_ids = lax.broadcasted_iota(jnp.int32, (F, F * E), 1)
    f_ids = lax.broadcasted_iota(jnp.int32, (F, F * E), 0)
    st = jnp.where(j_ids // E == f_ids, 1.0, 0.0).astype(jnp.float32)
    mxr = jnp.dot(mx, st, preferred_element_type=jnp.float32)

    a = mxr * x
    jj = lax.broadcasted_iota(jnp.int32, (F * E, E), 0)
    ee = lax.broadcasted_iota(jnp.int32, (F * E, E), 1)
    sm = jnp.where(jj % E == ee, 1.0, 0.0).astype(jnp.float32)
    sum_v = jnp.dot(a, sm, preferred_element_type=jnp.float32)
    sum_sq = jnp.dot(a * a, sm, preferred_element_type=jnp.float32)
    out_ref[...] = 0.5 * jnp.sum(sum_v * sum_v - sum_sq, axis=-1,
                                 keepdims=True)


_BB = 1024


def _dense(x, W1, b1, W2, b2, P):
    return pl.pallas_call(
        _dense_body,
        grid=(B // _BB,),
        in_specs=[
            pl.BlockSpec((_BB, F * E), lambda i: (i, 0)),
            pl.BlockSpec((F * E, L1), lambda i: (0, 0)),
            pl.BlockSpec((1, L1), lambda i: (0, 0)),
            pl.BlockSpec((L1, L2), lambda i: (0, 0)),
            pl.BlockSpec((1, L2), lambda i: (0, 0)),
            pl.BlockSpec((L2, F), lambda i: (0, 0)),
        ],
        out_specs=pl.BlockSpec((_BB, 1), lambda i: (i, 0)),
        out_shape=jax.ShapeDtypeStruct((B, 1), jnp.float32),
    )(x, W1, b1, W2, b2, P)


def kernel(sparse, dense, tables, W1, b1, W2, b2, P):
    tab = tables.transpose(0, 2, 1)
    idx = sparse.T.reshape(F * B)
    rows = _make_sc_gather()(tab, idx)
    x = rows[:, :E].reshape(B, F * E)
    return _dense(x, W1, b1.reshape(1, L1), W2, b2.reshape(1, L2), P)


# speedup vs baseline: 312.7931x; 1.0424x over previous
import functools

import jax
import jax.numpy as jnp
from jax import lax
from jax.experimental import pallas as pl
from jax.experimental.pallas import tpu as pltpu
from jax.experimental.pallas import tpu_sc as plsc

B = 4096
F = 26
V = 100000
E = 16
L1 = 256
L2 = 64

NCORE = 2
NSUB = 16
FPC = F // NCORE
VRANGE = V // NSUB
VMAIN = 6400
VBLOCK = 6432
SCHUNK = 64
EPAD = 128
NOUT = B * F


def _floor128(x):
    return (x // 128) * 128


@functools.cache
def _make_sc_gather():
    mesh = plsc.VectorSubcoreMesh(core_axis_name="c", subcore_axis_name="s")
    return functools.partial(
        pl.kernel,
        out_type=jax.ShapeDtypeStruct((NOUT, EPAD), jnp.float32),
        mesh=mesh,
        scratch_types=[
            pltpu.VMEM((E, VBLOCK), jnp.float32),
            pltpu.VMEM((B,), jnp.int32),
            pltpu.VMEM((B + 32,), jnp.int32),
            pltpu.VMEM((2, SCHUNK, EPAD), jnp.float32),
            pltpu.VMEM((2, SCHUNK), jnp.int32),
            pltpu.SemaphoreType.DMA,
            pltpu.SemaphoreType.DMA,
            pltpu.SemaphoreType.DMA,
        ],
        compiler_params=pltpu.CompilerParams(needs_layout_passes=False),
    )(_sc_gather_body)


def _sc_gather_body(tab_hbm, idx_hbm, out_hbm, block_v, idx_v, wl_v, stage_v,
                    didx_v, sem0, sem1, sem_slab):
    c = lax.axis_index("c")
    s = lax.axis_index("s")
    lane = lax.iota(jnp.int32, 16)

    lo = s * VRANGE
    load_lo = jnp.where(s == NSUB - 1, V - VBLOCK, _floor128(lo))
    load_lo = pl.multiple_of(load_lo, 128)

    def _field(k, carry):
        f = c * FPC + k

        slab_copy = pltpu.async_copy(tab_hbm.at[f, :, pl.ds(load_lo, VMAIN)],
                                     block_v.at[:, pl.ds(0, VMAIN)], sem_slab)
        pltpu.sync_copy(idx_hbm.at[pl.ds(f * B, B)], idx_v)

        def _scan(g4, count):
            for u in range(4):
                base_b = (g4 * 4 + u) * 16
                v = idx_v[pl.ds(base_b, 16)]
                mask = (v >= lo) & (v < lo + VRANGE)
                inc = plsc.all_reduce_population_count(mask)[0]
                plsc.store_compressed(wl_v.at[pl.ds(count, 16)],
                                      base_b + lane, mask=mask)
                count = count + inc
            return count

        count = lax.fori_loop(0, B // 64, _scan, 0)

        slab_copy.wait()

        @pl.when(s == NSUB - 1)
        def _():
            pltpu.sync_copy(tab_hbm.at[f, :, pl.ds(V - 32, 32)],
                            block_v.at[:, pl.ds(VMAIN, 32)])

        nb = (count + SCHUNK - 1) // SCHUNK
        par, o0, o1 = carry

        def _blk(blk, oc):
            b0_, b1_ = oc
            buf = (par + blk) % 2

            @pl.when((buf == 0) & (b0_ == 1))
            def _():
                pltpu.make_async_copy(stage_v.at[0],
                                      out_hbm.at[didx_v.at[0]],
                                      sem0).wait()

            @pl.when((buf == 1) & (b1_ == 1))
            def _():
                pltpu.make_async_copy(stage_v.at[1],
                                      out_hbm.at[didx_v.at[1]],
                                      sem1).wait()
            b0 = wl_v[pl.ds(blk * SCHUNK, 16)][0]
            for gg in range(SCHUNK // 16):
                slot = blk * SCHUNK + gg * 16 + lane
                sm = slot < count
                b = wl_v[pl.ds(blk * SCHUNK + gg * 16, 16)]
                b = jnp.where(sm, b, b0)
                vv = plsc.load_gather(idx_v, [b])
                cols = vv - load_lo
                rows = gg * 16 + lane
                for e in range(E):
                    erow = jnp.full((16,), e, jnp.int32)
                    got = plsc.load_gather(block_v, [erow, cols])
                    plsc.store_scatter(stage_v.at[buf], [rows, erow], got)
                didx_v[buf, pl.ds(gg * 16, 16)] = b * F + f

            @pl.when(buf == 0)
            def _():
                pltpu.async_copy(stage_v.at[0], out_hbm.at[didx_v.at[0]],
                                 sem0)

            @pl.when(buf == 1)
            def _():
                pltpu.async_copy(stage_v.at[1], out_hbm.at[didx_v.at[1]],
                                 sem1)
            return (jnp.where(buf == 0, 1, b0_), jnp.where(buf == 1, 1, b1_))

        o0, o1 = lax.fori_loop(0, nb, _blk, (o0, o1))
        return ((par + nb) % 2, o0, o1)

    par, o0, o1 = lax.fori_loop(0, FPC, _field,
                                (jnp.int32(0), jnp.int32(0), jnp.int32(0)))

    @pl.when(o0 == 1)
    def _():
        pltpu.make_async_copy(stage_v.at[0], out_hbm.at[didx_v.at[0]],
                              sem0).wait()

    @pl.when(o1 == 1)
    def _():
        pltpu.make_async_copy(stage_v.at[1], out_hbm.at[didx_v.at[1]],
                              sem1).wait()


def _dense_body(x_ref, w1_ref, b1_ref, w2_ref, b2_ref, p_ref, out_ref):
    x = x_ref[...]
    h1 = jnp.dot(x, w1_ref[...], preferred_element_type=jnp.float32)
    h1 = jnp.maximum(h1 + b1_ref[...], 0.0)
    ux = jnp.dot(h1, w2_ref[...], preferred_element_type=jnp.float32)
    ux = jnp.maximum(ux + b2_ref[...], 0.0)
    mx_ = jnp.dot(ux, p_ref[...], preferred_element_type=jnp.float32)
    m = jnp.max(mx_, axis=-1, keepdims=True)
    ex = jnp.exp(mx_ - m)
    mx = (jnp.float32(F) * ex) / jnp.sum(ex, axis=-1, keepdims=True)

    j_ids = lax.broadcasted_iota(jnp.int32, (F, F * E), 1)
    f_ids = lax.broadcasted_iota(jnp.int32, (F, F * E), 0)
    st = jnp.where(j_ids // E == f_ids, 1.0, 0.0).astype(jnp.float32)
    mxr = jnp.dot(mx, st, preferred_element_type=jnp.float32)

    a = mxr * x
    jj = lax.broadcasted_iota(jnp.int32, (F * E, E), 0)
    ee = lax.broadcasted_iota(jnp.int32, (F * E, E), 1)
    sm = jnp.where(jj % E == ee, 1.0, 0.0).astype(jnp.float32)
    sum_v = jnp.dot(a, sm, preferred_element_type=jnp.float32)
    sum_sq = jnp.dot(a * a, sm, preferred_element_type=jnp.float32)
    out_ref[...] = 0.5 * jnp.sum(sum_v * sum_v - sum_sq, axis=-1,
                                 keepdims=True)


_BB = 1024


def _dense(x, W1, b1, W2, b2, P):
    return pl.pallas_call(
        _dense_body,
        grid=(B // _BB,),
        in_specs=[
            pl.BlockSpec((_BB, F * E), lambda i: (i, 0)),
            pl.BlockSpec((F * E, L1), lambda i: (0, 0)),
            pl.BlockSpec((1, L1), lambda i: (0, 0)),
            pl.BlockSpec((L1, L2), lambda i: (0, 0)),
            pl.BlockSpec((1, L2), lambda i: (0, 0)),
            pl.BlockSpec((L2, F), lambda i: (0, 0)),
        ],
        out_specs=pl.BlockSpec((_BB, 1), lambda i: (i, 0)),
        out_shape=jax.ShapeDtypeStruct((B, 1), jnp.float32),
    )(x, W1, b1, W2, b2, P)


def kernel(sparse, dense, tables, W1, b1, W2, b2, P):
    tab = tables.transpose(0, 2, 1)
    idx = sparse.T.reshape(F * B)
    rows = _make_sc_gather()(tab, idx)
    x = rows[:, :E].reshape(B, F * E)
    return _dense(x, W1, b1.reshape(1, L1), W2, b2.reshape(1, L2), P)
